# speedup vs baseline: 3.3130x; 3.3130x over previous
import functools

import jax
import jax.numpy as jnp
from jax import lax
from jax.experimental import pallas as pl
from jax.experimental.pallas import tpu as pltpu
from jax.experimental.pallas import tpu_sc as plsc

_G = 128
_NBUF = 2


@functools.lru_cache(maxsize=None)
def _build(num_idx, vocab, embed):
    info = plsc.get_sparse_core_info()
    nc, ns = info.num_cores, info.num_subcores
    nw = nc * ns
    assert num_idx % (nw * _G) == 0
    per_w = num_idx // nw
    nchunk = per_w // _G

    mesh = plsc.VectorSubcoreMesh(core_axis_name="c", subcore_axis_name="s")

    @functools.partial(
        pl.kernel,
        out_type=jax.ShapeDtypeStruct((num_idx, embed), jnp.float32),
        mesh=mesh,
        scratch_types=[
            pltpu.VMEM((per_w,), jnp.int32),
            pltpu.VMEM((_G, embed), jnp.float32),
            pltpu.VMEM((_G, embed), jnp.float32),
            pltpu.SemaphoreType.DMA,
            pltpu.SemaphoreType.DMA,
        ],
    )
    def gather_kernel(idx_hbm, table_hbm, out_hbm, idx_v, buf0, buf1, s0, s1):
        wid = lax.axis_index("s") * nc + lax.axis_index("c")
        base = wid * per_w
        pltpu.sync_copy(idx_hbm.at[pl.ds(base, per_w)], idx_v)

        bufs = (buf0, buf1)
        sems = (s0, s1)

        def start(chunk, b):
            pltpu.async_copy(
                table_hbm.at[idx_v.at[pl.ds(chunk * _G, _G)]], bufs[b], sems[b]
            )

        def finish(chunk, b):
            pltpu.make_async_copy(table_hbm.at[pl.ds(0, _G)], bufs[b], sems[b]).wait()
            pltpu.sync_copy(bufs[b], out_hbm.at[pl.ds(base + chunk * _G, _G)])

        for b in range(_NBUF):
            start(b, b)

        @pl.loop(0, nchunk - _NBUF, step=_NBUF)
        def _(g):
            for b in range(_NBUF):
                finish(g + b, b)
                start(g + b + _NBUF, b)

        for b in range(_NBUF):
            finish(nchunk - _NBUF + b, b)

    return gather_kernel


def kernel(data, ivectors):
    b, t = data.shape
    vocab, embed = ivectors.shape
    idx = data.reshape(-1).astype(jnp.int32)
    out = _build(idx.shape[0], vocab, embed)(idx, ivectors)
    return out.reshape(b, t, embed)


# speedup vs baseline: 3.3308x; 1.0054x over previous
import functools

import jax
import jax.numpy as jnp
from jax import lax
from jax.experimental import pallas as pl
from jax.experimental.pallas import tpu as pltpu
from jax.experimental.pallas import tpu_sc as plsc

_G = 128
_D = 4


@functools.lru_cache(maxsize=None)
def _build(num_idx, vocab, embed):
    info = plsc.get_sparse_core_info()
    nc, ns = info.num_cores, info.num_subcores
    nw = nc * ns
    assert num_idx % (nw * _G) == 0
    per_w = num_idx // nw
    nchunk = per_w // _G
    assert nchunk > 2 * _D and (nchunk - 2 - 4) % _D == 0

    mesh = plsc.VectorSubcoreMesh(core_axis_name="c", subcore_axis_name="s")

    @functools.partial(
        pl.kernel,
        out_type=jax.ShapeDtypeStruct((num_idx, embed), jnp.float32),
        mesh=mesh,
        scratch_types=[
            pltpu.VMEM((per_w,), jnp.int32),
        ]
        + [pltpu.VMEM((_G, embed), jnp.float32) for _ in range(_D)]
        + [pltpu.SemaphoreType.DMA for _ in range(2 * _D)],
    )
    def gather_kernel(idx_hbm, table_hbm, out_hbm, idx_v, *rest):
        bufs = rest[:_D]
        gsem = rest[_D : 2 * _D]
        ssem = rest[2 * _D :]

        wid = lax.axis_index("s") * nc + lax.axis_index("c")
        base = wid * per_w
        pltpu.sync_copy(idx_hbm.at[pl.ds(base, per_w)], idx_v)

        def gather(c, b):
            pltpu.async_copy(
                table_hbm.at[idx_v.at[pl.ds(c * _G, _G)]], bufs[b], gsem[b]
            )

        def store(c, b):
            pltpu.async_copy(bufs[b], out_hbm.at[pl.ds(base + c * _G, _G)], ssem[b])

        def wait_gather(b):
            pltpu.make_async_copy(table_hbm.at[pl.ds(0, _G)], bufs[b], gsem[b]).wait()

        def wait_store(b):
            pltpu.make_async_copy(bufs[b], out_hbm.at[pl.ds(base, _G)], ssem[b]).wait()

        gather(0, 0)
        gather(1, 1)
        for c in (0, 1):
            gather(c + 2, (c + 2) % _D)
            wait_gather(c % _D)
            store(c, c % _D)

        @pl.loop(2, nchunk - _D, step=_D)
        def _(g):
            for j in range(_D):
                c = g + j
                bc = (2 + j) % _D
                bg = (bc + 2) % _D
                wait_store(bg)
                gather(c + 2, bg)
                wait_gather(bc)
                store(c, bc)

        for c in (nchunk - 4, nchunk - 3):
            bc, bg = c % _D, (c + 2) % _D
            wait_store(bg)
            gather(c + 2, bg)
            wait_gather(bc)
            store(c, bc)
        for c in (nchunk - 2, nchunk - 1):
            wait_gather(c % _D)
            store(c, c % _D)
        for b in range(_D):
            wait_store(b)

    return gather_kernel


def kernel(data, ivectors):
    b, t = data.shape
    vocab, embed = ivectors.shape
    idx = data.reshape(-1).astype(jnp.int32)
    out = _build(idx.shape[0], vocab, embed)(idx, ivectors)
    return out.reshape(b, t, embed)


# speedup vs baseline: 5.9533x; 1.7874x over previous
import functools

import jax
import jax.numpy as jnp
from jax import lax
from jax.experimental import pallas as pl
from jax.experimental.pallas import tpu as pltpu
from jax.experimental.pallas import tpu_sc as plsc

_NB = 4
_D = 4
_TPAD = 64


@functools.lru_cache(maxsize=None)
def _build(batch, seq, vocab, embed):
    info = plsc.get_sparse_core_info()
    nc, ns = info.num_cores, info.num_subcores
    nw = nc * ns
    assert batch % (nw * _NB) == 0
    b_per_w = batch // nw
    nchunk = b_per_w // _NB
    lag = _D // 2
    head = lag
    mid = ((nchunk - head - lag) // _D) * _D
    assert mid > 0

    mesh = plsc.VectorSubcoreMesh(core_axis_name="c", subcore_axis_name="s")

    @functools.partial(
        pl.kernel,
        out_type=jax.ShapeDtypeStruct((batch, seq, embed), jnp.float32),
        mesh=mesh,
        scratch_types=[
            pltpu.VMEM((b_per_w * _TPAD,), jnp.int32),
        ]
        + [pltpu.VMEM((_NB, seq, embed), jnp.float32) for _ in range(_D)]
        + [pltpu.SemaphoreType.DMA for _ in range(2 * _D)],
    )
    def gather_kernel(idx_hbm, table_hbm, out_hbm, idx_v, *rest):
        bufs = rest[:_D]
        gsem = rest[_D : 2 * _D]
        ssem = rest[2 * _D :]

        wid = lax.axis_index("s") * nc + lax.axis_index("c")
        base = wid * b_per_w
        pltpu.sync_copy(idx_hbm.at[pl.ds(base * _TPAD, b_per_w * _TPAD)], idx_v)

        def gather(c, b):
            for i in range(_NB):
                pltpu.async_copy(
                    table_hbm.at[idx_v.at[pl.ds((c * _NB + i) * _TPAD, seq)]],
                    bufs[b].at[i],
                    gsem[b],
                )

        def store(c, b):
            pltpu.async_copy(
                bufs[b], out_hbm.at[pl.ds(base + c * _NB, _NB)], ssem[b]
            )

        def wait_gather(b):
            pltpu.make_async_copy(
                out_hbm.at[pl.ds(base, _NB)], bufs[b], gsem[b]
            ).wait()

        def wait_store(b):
            pltpu.make_async_copy(
                bufs[b], out_hbm.at[pl.ds(base, _NB)], ssem[b]
            ).wait()

        def body(c, bc, with_gather, with_wait_store):
            if with_gather:
                bg = (bc + lag) % _D
                if with_wait_store:
                    wait_store(bg)
                gather(c + lag, bg)
            wait_gather(bc)
            store(c, bc)

        for c in range(lag):
            gather(c, c % _D)
        for c in range(head):
            body(c, c % _D, True, c + lag >= _D)

        @pl.loop(head, head + mid, step=_D)
        def _(g):
            for j in range(_D):
                body(g + j, (head + j) % _D, True, True)

        for c in range(head + mid, nchunk):
            if c + lag < nchunk:
                body(c, c % _D, True, True)
            else:
                body(c, c % _D, False, False)
        for b in range(_D):
            wait_store(b)

    return gather_kernel


def kernel(data, ivectors):
    b, t = data.shape
    vocab, embed = ivectors.shape
    idx = jnp.pad(data.astype(jnp.int32), ((0, 0), (0, _TPAD - t))).reshape(-1)
    return _build(b, t, vocab, embed)(idx, ivectors)


# speedup vs baseline: 5.9533x; 1.0000x over previous
import functools

import jax
import jax.numpy as jnp
from jax import lax
from jax.experimental import pallas as pl
from jax.experimental.pallas import tpu as pltpu
from jax.experimental.pallas import tpu_sc as plsc

_NB = 4
_D = 4
_TPAD = 64


@functools.lru_cache(maxsize=None)
def _build(batch, seq, vocab, embed):
    info = plsc.get_sparse_core_info()
    nc, ns = info.num_cores, info.num_subcores
    nw = nc * ns
    assert batch % (nw * _NB) == 0
    b_per_w = batch // nw
    nchunk = b_per_w // _NB
    lag = _D // 2
    head = lag
    mid = ((nchunk - head - lag) // _D) * _D
    assert mid > 0

    mesh = plsc.VectorSubcoreMesh(core_axis_name="c", subcore_axis_name="s")

    @functools.partial(
        pl.kernel,
        out_type=jax.ShapeDtypeStruct((batch, seq, embed), jnp.float32),
        mesh=mesh,
        compiler_params=pltpu.CompilerParams(use_tc_tiling_on_sc=True),
        scratch_types=[
            pltpu.VMEM((b_per_w * _TPAD,), jnp.int32),
        ]
        + [pltpu.VMEM((_NB, seq, embed), jnp.float32) for _ in range(_D)]
        + [pltpu.SemaphoreType.DMA for _ in range(2 * _D)],
    )
    def gather_kernel(idx_hbm, table_hbm, out_hbm, idx_v, *rest):
        bufs = rest[:_D]
        gsem = rest[_D : 2 * _D]
        ssem = rest[2 * _D :]

        wid = lax.axis_index("s") * nc + lax.axis_index("c")
        base = wid * b_per_w
        pltpu.sync_copy(idx_hbm.at[pl.ds(base * _TPAD, b_per_w * _TPAD)], idx_v)

        def gather(c, b):
            for i in range(_NB):
                pltpu.async_copy(
                    table_hbm.at[idx_v.at[pl.ds((c * _NB + i) * _TPAD, seq)]],
                    bufs[b].at[i],
                    gsem[b],
                )

        def store(c, b):
            pltpu.async_copy(
                bufs[b], out_hbm.at[pl.ds(base + c * _NB, _NB)], ssem[b]
            )

        def wait_gather(b):
            pltpu.make_async_copy(
                out_hbm.at[pl.ds(base, _NB)], bufs[b], gsem[b]
            ).wait()

        def wait_store(b):
            pltpu.make_async_copy(
                bufs[b], out_hbm.at[pl.ds(base, _NB)], ssem[b]
            ).wait()

        def body(c, bc, with_gather, with_wait_store):
            if with_gather:
                bg = (bc + lag) % _D
                if with_wait_store:
                    wait_store(bg)
                gather(c + lag, bg)
            wait_gather(bc)
            store(c, bc)

        for c in range(lag):
            gather(c, c % _D)
        for c in range(head):
            body(c, c % _D, True, c + lag >= _D)

        @pl.loop(head, head + mid, step=_D)
        def _(g):
            for j in range(_D):
                body(g + j, (head + j) % _D, True, True)

        for c in range(head + mid, nchunk):
            if c + lag < nchunk:
                body(c, c % _D, True, True)
            else:
                body(c, c % _D, False, False)
        for b in range(_D):
            wait_store(b)

    return gather_kernel


def kernel(data, ivectors):
    b, t = data.shape
    vocab, embed = ivectors.shape
    idx = jnp.pad(data.astype(jnp.int32), ((0, 0), (0, _TPAD - t))).reshape(-1)
    return _build(b, t, vocab, embed)(idx, ivectors)


# speedup vs baseline: 10.5873x; 1.7784x over previous
import functools

import jax
import jax.numpy as jnp
from jax import lax
from jax.experimental import pallas as pl
from jax.experimental.pallas import tpu as pltpu
from jax.experimental.pallas import tpu_sc as plsc

_G = 128
_D = 4


@functools.lru_cache(maxsize=None)
def _build(batch, seq, vocab, embed):
    info = plsc.get_sparse_core_info()
    nc, ns = info.num_cores, info.num_subcores
    nw = nc * ns
    assert batch % (nw * _G) == 0
    num_idx = batch * seq
    per_w = num_idx // nw
    nchunk = seq
    lag = _D // 2
    head = lag
    mid = ((nchunk - head - lag) // _D) * _D
    assert mid > 0

    mesh = plsc.VectorSubcoreMesh(core_axis_name="c", subcore_axis_name="s")

    @functools.partial(
        pl.kernel,
        out_type=jax.ShapeDtypeStruct((num_idx, embed), jnp.float32),
        mesh=mesh,
        scratch_types=[
            pltpu.VMEM((per_w,), jnp.int32),
        ]
        + [pltpu.VMEM((_G, embed), jnp.float32) for _ in range(_D)]
        + [pltpu.SemaphoreType.DMA for _ in range(2 * _D)],
    )
    def gather_kernel(idx_hbm, table_hbm, out_hbm, idx_v, *rest):
        bufs = rest[:_D]
        gsem = rest[_D : 2 * _D]
        ssem = rest[2 * _D :]

        wid = lax.axis_index("s") * nc + lax.axis_index("c")
        pltpu.sync_copy(idx_hbm.at[pl.ds(wid * per_w, per_w)], idx_v)
        obase = wid * _G

        def gather(c, b):
            pltpu.async_copy(
                table_hbm.at[idx_v.at[pl.ds(c * _G, _G)]], bufs[b], gsem[b]
            )

        def store(c, b):
            pltpu.async_copy(
                bufs[b], out_hbm.at[pl.ds(c * batch + obase, _G)], ssem[b]
            )

        def wait_gather(b):
            pltpu.make_async_copy(
                out_hbm.at[pl.ds(obase, _G)], bufs[b], gsem[b]
            ).wait()

        def wait_store(b):
            pltpu.make_async_copy(
                bufs[b], out_hbm.at[pl.ds(obase, _G)], ssem[b]
            ).wait()

        def body(c, bc, with_gather, with_wait_store):
            if with_gather:
                bg = (bc + lag) % _D
                if with_wait_store:
                    wait_store(bg)
                gather(c + lag, bg)
            wait_gather(bc)
            store(c, bc)

        for c in range(lag):
            gather(c, c % _D)
        for c in range(head):
            body(c, c % _D, True, c + lag >= _D)

        @pl.loop(head, head + mid, step=_D)
        def _(g):
            for j in range(_D):
                body(g + j, (head + j) % _D, True, True)

        for c in range(head + mid, nchunk):
            if c + lag < nchunk:
                body(c, c % _D, True, True)
            else:
                body(c, c % _D, False, False)
        for b in range(_D):
            wait_store(b)

    return gather_kernel


def kernel(data, ivectors):
    b, t = data.shape
    vocab, embed = ivectors.shape
    nw = 32
    idx = (
        data.astype(jnp.int32)
        .reshape(nw, b // nw, t)
        .transpose(0, 2, 1)
        .reshape(-1)
    )
    out = _build(b, t, vocab, embed)(idx, ivectors)
    return out.reshape(t, b, embed).transpose(1, 0, 2)


# speedup vs baseline: 10.6965x; 1.0103x over previous
import functools

import jax
import jax.numpy as jnp
from jax import lax
from jax.experimental import pallas as pl
from jax.experimental.pallas import tpu as pltpu
from jax.experimental.pallas import tpu_sc as plsc

_G = 128
_D = 6


@functools.lru_cache(maxsize=None)
def _build(batch, seq, vocab, embed):
    info = plsc.get_sparse_core_info()
    nc, ns = info.num_cores, info.num_subcores
    nw = nc * ns
    assert batch % (nw * _G) == 0
    num_idx = batch * seq
    per_w = num_idx // nw
    nchunk = seq
    lag = _D // 2
    head = lag
    mid = ((nchunk - head - lag) // _D) * _D
    assert mid > 0

    mesh = plsc.VectorSubcoreMesh(core_axis_name="c", subcore_axis_name="s")

    @functools.partial(
        pl.kernel,
        out_type=jax.ShapeDtypeStruct((num_idx, embed), jnp.float32),
        mesh=mesh,
        scratch_types=[
            pltpu.VMEM((per_w,), jnp.int32),
        ]
        + [pltpu.VMEM((_G, embed), jnp.float32) for _ in range(_D)]
        + [pltpu.SemaphoreType.DMA for _ in range(2 * _D)],
    )
    def gather_kernel(idx_hbm, table_hbm, out_hbm, idx_v, *rest):
        bufs = rest[:_D]
        gsem = rest[_D : 2 * _D]
        ssem = rest[2 * _D :]

        wid = lax.axis_index("s") * nc + lax.axis_index("c")
        pltpu.sync_copy(idx_hbm.at[pl.ds(wid * per_w, per_w)], idx_v)
        obase = wid * _G

        def gather(c, b):
            pltpu.async_copy(
                table_hbm.at[idx_v.at[pl.ds(c * _G, _G)]], bufs[b], gsem[b]
            )

        def store(c, b):
            pltpu.async_copy(
                bufs[b], out_hbm.at[pl.ds(c * batch + obase, _G)], ssem[b]
            )

        def wait_gather(b):
            pltpu.make_async_copy(
                out_hbm.at[pl.ds(obase, _G)], bufs[b], gsem[b]
            ).wait()

        def wait_store(b):
            pltpu.make_async_copy(
                bufs[b], out_hbm.at[pl.ds(obase, _G)], ssem[b]
            ).wait()

        def body(c, bc, with_gather, with_wait_store):
            if with_gather:
                bg = (bc + lag) % _D
                if with_wait_store:
                    wait_store(bg)
                gather(c + lag, bg)
            wait_gather(bc)
            store(c, bc)

        for c in range(lag):
            gather(c, c % _D)
        for c in range(head):
            body(c, c % _D, True, c + lag >= _D)

        @pl.loop(head, head + mid, step=_D)
        def _(g):
            for j in range(_D):
                body(g + j, (head + j) % _D, True, True)

        for c in range(head + mid, nchunk):
            if c + lag < nchunk:
                body(c, c % _D, True, True)
            else:
                body(c, c % _D, False, False)
        for b in range(_D):
            wait_store(b)

    return gather_kernel


def kernel(data, ivectors):
    b, t = data.shape
    vocab, embed = ivectors.shape
    nw = 32
    idx = (
        data.astype(jnp.int32)
        .reshape(nw, b // nw, t)
        .transpose(0, 2, 1)
        .reshape(-1)
    )
    out = _build(b, t, vocab, embed)(idx, ivectors)
    return out.reshape(t, b, embed).transpose(1, 0, 2)
